# baseline (speedup 1.0000x reference)
import functools

import jax
import jax.numpy as jnp
from jax.experimental import pallas as pl
from jax.experimental.pallas import tpu as pltpu

_K = 16
_CHUNK = 1024
_SUB = 128
_IMAX = 2147483647
_IMIN = -2147483648


def _knn_body(nkeys, q_ref, k_ref, dv_ref, di_ref, d_scr, m_scr, rv_scr,
              ri_scr):
    c = pl.program_id(0)
    nc = pl.num_programs(0)
    nq = q_ref.shape[1]
    cw = k_ref.shape[0]
    nsub = cw // _SUB

    @pl.when(c == 0)
    def _init():
        rv_scr[...] = jnp.full((_K, nq), jnp.inf, dtype=jnp.float32)
        ri_scr[...] = -(jax.lax.broadcasted_iota(jnp.int32, (_K, nq), 0) + 1)

    qt = q_ref[...]
    k = k_ref[...]
    qn = jnp.sum(qt * qt, axis=0, keepdims=True)
    kn = jnp.sum(k * k, axis=1, keepdims=True)
    gcol = c * cw + jax.lax.broadcasted_iota(jnp.int32, (cw, 1), 0)
    kn = jnp.where(gcol < nkeys, kn, jnp.inf)
    d = -2.0 * jax.lax.dot_general(
        k, qt, (((1,), (0,)), ((), ())),
        preferred_element_type=jnp.float32)
    d = d + qn
    d = d + kn
    d_scr[...] = d
    m_scr[...] = jnp.min(d.reshape(nsub, _SUB, nq), axis=1)

    def _merge(_):
        mm = m_scr[...]
        m = jnp.min(mm, axis=0, keepdims=True)
        thr = jnp.max(rv_scr[...], axis=0, keepdims=True)
        need = m < thr
        siota = jax.lax.broadcasted_iota(jnp.int32, (nsub, nq), 0)
        s_sel = jnp.min(jnp.where(mm == m, siota, nsub), axis=0,
                        keepdims=True)
        s_star = jnp.min(jnp.where(need, s_sel, nsub))
        s_star = jnp.minimum(s_star, nsub - 1)
        serve = need & (s_sel == s_star)

        dd = d_scr[pl.ds(s_star * _SUB, _SUB), :]
        gb = (c * cw + s_star * _SUB +
              jax.lax.broadcasted_iota(jnp.int32, (_SUB, 1), 0))
        gbt = jnp.broadcast_to(gb, dd.shape)
        gsel = jnp.min(jnp.where((dd == m) & serve, gbt, _IMAX), axis=0,
                       keepdims=True)
        masked = jnp.where((gbt == gsel) & serve, jnp.inf, dd)
        d_scr[pl.ds(s_star * _SUB, _SUB), :] = masked
        m_scr[pl.ds(s_star, 1), :] = jnp.min(masked, axis=0, keepdims=True)

        rvv = rv_scr[...]
        rii = ri_scr[...]
        evict = jnp.max(jnp.where(rvv == thr, rii, _IMIN), axis=0,
                        keepdims=True)
        sel = (rii == evict) & serve
        rv_scr[...] = jnp.where(sel, jnp.broadcast_to(m, rvv.shape), rvv)
        ri_scr[...] = jnp.where(sel, jnp.broadcast_to(gsel, rii.shape), rii)
        return jnp.max(need.astype(jnp.int32)) > 0

    jax.lax.while_loop(lambda go: go, _merge, True)

    @pl.when(c == nc - 1)
    def _finalize():
        vals = rv_scr[...]
        idxs = ri_scr[...]
        vrows = []
        irows = []
        for _ in range(_K):
            m = jnp.min(vals, axis=0, keepdims=True)
            gs = jnp.min(jnp.where(vals == m, idxs, _IMAX), axis=0,
                         keepdims=True)
            vrows.append(m)
            irows.append(gs)
            vals = jnp.where(idxs == gs, jnp.inf, vals)
        dv_ref[...] = jnp.concatenate(vrows, axis=0)
        di_ref[...] = jnp.concatenate(irows, axis=0)


def _knn(queries, keys, chunk):
    nq, dim = queries.shape
    nk = keys.shape[0]
    nc = pl.cdiv(nk, chunk)
    npad = nc * chunk - nk
    keys_p = jnp.pad(keys, ((0, npad), (0, 0))) if npad else keys
    qt = queries.T
    dv, di = pl.pallas_call(
        functools.partial(_knn_body, nk),
        grid=(nc,),
        in_specs=[
            pl.BlockSpec((dim, nq), lambda c: (0, 0)),
            pl.BlockSpec((chunk, dim), lambda c: (c, 0)),
        ],
        out_specs=[
            pl.BlockSpec((_K, nq), lambda c: (0, 0)),
            pl.BlockSpec((_K, nq), lambda c: (0, 0)),
        ],
        out_shape=[
            jax.ShapeDtypeStruct((_K, nq), jnp.float32),
            jax.ShapeDtypeStruct((_K, nq), jnp.int32),
        ],
        scratch_shapes=[
            pltpu.VMEM((chunk, nq), jnp.float32),
            pltpu.VMEM((chunk // _SUB, nq), jnp.float32),
            pltpu.VMEM((_K, nq), jnp.float32),
            pltpu.VMEM((_K, nq), jnp.int32),
        ],
        compiler_params=pltpu.CompilerParams(
            dimension_semantics=("arbitrary",)),
    )(qt, keys_p)
    return dv.T, di.T


def kernel(queries, keys):
    dists, idx = _knn(queries, keys, _CHUNK)
    return (dists, idx)


# speedup vs baseline: 2.4667x; 2.4667x over previous
import functools

import jax
import jax.numpy as jnp
from jax import lax
from jax.experimental import pallas as pl
from jax.experimental.pallas import tpu as pltpu
from jax.experimental.pallas import tpu_sc as plsc

_K = 16
_CHUNK = 512
_SUB = 128
_L = 16
_IMAX = 2147483647
_IMIN = -2147483648


def _tc_body(nkeys, q_ref, k_ref, d_ref, cm_ref):
    c = pl.program_id(0)
    nq = q_ref.shape[0]
    cw = k_ref.shape[0]
    q = q_ref[...]
    k = k_ref[...]
    qn = jnp.sum(q * q, axis=1, keepdims=True)
    kn = jnp.sum(k * k, axis=1)[None, :]
    gidx = c * cw + jax.lax.broadcasted_iota(jnp.int32, (1, cw), 1)
    kn = jnp.where(gidx < nkeys, kn, jnp.inf)
    d = -2.0 * jnp.dot(q, k.T, preferred_element_type=jnp.float32)
    d = d + qn
    d = d + kn
    d_ref[...] = d
    cm_ref[...] = jnp.min(d.reshape(nq, cw // _SUB, _SUB),
                          axis=2)[None, :, :]


def _tc_distances(queries, keys_p, nkeys):
    nq, dim = queries.shape
    nkp = keys_p.shape[0]
    nc = nkp // _CHUNK
    return pl.pallas_call(
        functools.partial(_tc_body, nkeys),
        grid=(nc,),
        in_specs=[
            pl.BlockSpec((nq, dim), lambda c: (0, 0)),
            pl.BlockSpec((_CHUNK, dim), lambda c: (c, 0)),
        ],
        out_specs=[
            pl.BlockSpec((nq, _CHUNK), lambda c: (0, c)),
            pl.BlockSpec((1, nq, _CHUNK // _SUB), lambda c: (c, 0, 0)),
        ],
        out_shape=[
            jax.ShapeDtypeStruct((nq, nkp), jnp.float32),
            jax.ShapeDtypeStruct((nc, nq, _CHUNK // _SUB), jnp.float32),
        ],
        compiler_params=pltpu.CompilerParams(
            dimension_semantics=("arbitrary",)),
    )(queries, keys_p)


def _sc_topk(d_hbm_arr, cm_flat, nq, ncm):
    info = plsc.get_sparse_core_info()
    ncores = info.num_cores
    nworkers = ncores * info.num_subcores
    rows_per_w = nq // nworkers
    nv = ncm // _L
    mesh = plsc.VectorSubcoreMesh(core_axis_name="c", subcore_axis_name="s")

    @functools.partial(
        pl.kernel,
        mesh=mesh,
        out_type=[
            jax.ShapeDtypeStruct((nq * _K,), jnp.float32),
            jax.ShapeDtypeStruct((nq * _K,), jnp.int32),
        ],
        scratch_types=[
            pltpu.VMEM((ncm,), jnp.float32),
            pltpu.VMEM((8, _SUB), jnp.float32),
            pltpu.SMEM((ncm + 16,), jnp.int32),
            pltpu.VMEM((_K,), jnp.float32),
            pltpu.VMEM((_K,), jnp.int32),
        ],
        compiler_params=pltpu.CompilerParams(needs_layout_passes=False),
    )
    def sc_kern(d_hbm, cm_hbm, ov_hbm, oi_hbm, cmb, dtile, clist, ovb, oib):
        wid = lax.axis_index("s") * ncores + lax.axis_index("c")
        lane = jax.lax.broadcasted_iota(jnp.int32, (_L,), 0)
        inf16 = jnp.full((_L,), jnp.inf, dtype=jnp.float32)

        def row_body(j, _):
            row = wid * rows_per_w + j
            rbase = (row // 8) * 8
            rr = row % 8
            pltpu.sync_copy(cm_hbm.at[pl.ds(row * ncm, ncm)], cmb)

            def u_vreg(i, uv):
                v = cmb[pl.ds(i * _L, _L)]

                def ins_cond(c):
                    uvc, done = c
                    return jnp.any((v < jnp.max(uvc)) & ~done)

                def ins_body(c):
                    uvc, done = c
                    thr = jnp.max(uvc)
                    m = (v < thr) & ~done
                    l = jnp.max(plsc.all_reduce_ffs(m))
                    sel = lane == l
                    vl = jnp.max(jnp.where(sel, v, -jnp.inf))
                    lm = jnp.max(plsc.all_reduce_ffs(uvc == thr))
                    uvc = jnp.where(lane == lm, vl, uvc)
                    return (uvc, done | sel)

                uv, _ = lax.while_loop(
                    ins_cond, ins_body,
                    (uv, jnp.zeros((_L,), dtype=jnp.bool_)))
                return uv

            uvec = lax.fori_loop(0, nv, u_vreg, inf16)
            u = jnp.max(uvec)

            def cand_vreg(i, cnt):
                v = cmb[pl.ds(i * _L, _L)]
                cand = v <= u

                def c_cond(c):
                    _, msk = c
                    return jnp.any(msk)

                def c_body(c):
                    cn, msk = c
                    l = jnp.max(plsc.all_reduce_ffs(msk))
                    clist[cn] = i * _L + l
                    return (cn + 1, msk & ~(lane == l))

                cnt, _ = lax.while_loop(c_cond, c_body, (cnt, cand))
                return cnt

            ncand = lax.fori_loop(0, nv, cand_vreg, jnp.int32(0))

            rv0 = inf16
            ri0 = -(lane + 1)

            def scan_cand(ci, carry):
                rv, ri = carry
                cid = clist[ci]
                pltpu.sync_copy(
                    d_hbm.at[pl.ds(rbase, 8), pl.ds(cid * _SUB, _SUB)],
                    dtile)

                def scan_vreg(t, carry2):
                    rv2, ri2 = carry2
                    dv = dtile[rr, pl.ds(t * _L, _L)]
                    gi = cid * _SUB + t * _L + lane
                    hit = dv <= jnp.max(rv2)

                    def h_cond(c):
                        _, _, msk = c
                        return jnp.any(msk)

                    def h_body(c):
                        rv3, ri3, msk = c
                        l = jnp.max(plsc.all_reduce_ffs(msk))
                        sel = lane == l
                        ve = jnp.max(jnp.where(sel, dv, -jnp.inf))
                        ie = jnp.max(jnp.where(sel, gi, _IMIN))
                        thr = jnp.max(rv3)
                        ei = jnp.max(jnp.where(rv3 == thr, ri3, _IMIN))
                        better = (ve < thr) | ((ve == thr) & (ie < ei))
                        selm = (rv3 == thr) & (ri3 == ei) & better
                        rv3 = jnp.where(selm, ve, rv3)
                        ri3 = jnp.where(selm, ie, ri3)
                        return (rv3, ri3, msk & ~sel)

                    rv2, ri2, _ = lax.while_loop(h_cond, h_body,
                                                 (rv2, ri2, hit))
                    return (rv2, ri2)

                return lax.fori_loop(0, _SUB // _L, scan_vreg, (rv, ri))

            rv, ri = lax.fori_loop(0, ncand, scan_cand, (rv0, ri0))

            ov = jnp.zeros((_L,), dtype=jnp.float32)
            oi = jnp.zeros((_L,), dtype=jnp.int32)
            for kk in range(_K):
                mval = jnp.min(rv)
                mi = jnp.min(jnp.where(rv == mval, ri, _IMAX))
                ksel = lane == kk
                ov = jnp.where(ksel, mval, ov)
                oi = jnp.where(ksel, mi, oi)
                rv = jnp.where((rv == mval) & (ri == mi), jnp.inf, rv)
            ovb[...] = ov
            oib[...] = oi
            pltpu.sync_copy(ovb, ov_hbm.at[pl.ds(row * _K, _K)])
            pltpu.sync_copy(oib, oi_hbm.at[pl.ds(row * _K, _K)])
            return 0

        lax.fori_loop(0, rows_per_w, row_body, 0)

    return sc_kern(d_hbm_arr, cm_flat)


def kernel(queries, keys):
    nq, dim = queries.shape
    nk = keys.shape[0]
    nc = pl.cdiv(nk, _CHUNK)
    npad = nc * _CHUNK - nk
    keys_p = jnp.pad(keys, ((0, npad), (0, 0))) if npad else keys
    d_hbm, cm3 = _tc_distances(queries, keys_p, nk)
    ncm = cm3.shape[0] * cm3.shape[2]
    cm = jnp.transpose(cm3, (1, 0, 2)).reshape(nq * ncm)
    ovf, oif = _sc_topk(d_hbm, cm, nq, ncm)
    return (ovf.reshape(nq, _K), oif.reshape(nq, _K))


# speedup vs baseline: 3.2908x; 1.3341x over previous
import functools

import jax
import jax.numpy as jnp
from jax import lax
from jax.experimental import pallas as pl
from jax.experimental.pallas import tpu as pltpu
from jax.experimental.pallas import tpu_sc as plsc

_K = 16
_CHUNK = 512
_SUB = 128
_L = 16
_IMAX = 2147483647
_IMIN = -2147483648


def _tc_body(nkeys, q_ref, k_ref, d_ref, cm_ref):
    c = pl.program_id(0)
    nq = q_ref.shape[0]
    cw = k_ref.shape[0]
    q = q_ref[...]
    k = k_ref[...]
    qn = jnp.sum(q * q, axis=1, keepdims=True)
    kn = jnp.sum(k * k, axis=1)[None, :]
    gidx = c * cw + jax.lax.broadcasted_iota(jnp.int32, (1, cw), 1)
    kn = jnp.where(gidx < nkeys, kn, jnp.inf)
    d = -2.0 * jnp.dot(q, k.T, preferred_element_type=jnp.float32)
    d = d + qn
    d = d + kn
    d_ref[...] = d
    cm_ref[...] = jnp.min(d.reshape(nq, cw // _SUB, _SUB),
                          axis=2)[None, :, :]


def _tc_distances(queries, keys_p, nkeys):
    nq, dim = queries.shape
    nkp = keys_p.shape[0]
    nc = nkp // _CHUNK
    return pl.pallas_call(
        functools.partial(_tc_body, nkeys),
        grid=(nc,),
        in_specs=[
            pl.BlockSpec((nq, dim), lambda c: (0, 0)),
            pl.BlockSpec((_CHUNK, dim), lambda c: (c, 0)),
        ],
        out_specs=[
            pl.BlockSpec((nq, _CHUNK), lambda c: (0, c)),
            pl.BlockSpec((1, nq, _CHUNK // _SUB), lambda c: (c, 0, 0)),
        ],
        out_shape=[
            jax.ShapeDtypeStruct((nq, nkp), jnp.float32),
            jax.ShapeDtypeStruct((nc, nq, _CHUNK // _SUB), jnp.float32),
        ],
        compiler_params=pltpu.CompilerParams(
            dimension_semantics=("arbitrary",)),
    )(queries, keys_p)


def _sc_topk(d_hbm_arr, cm_flat, nq, ncm):
    info = plsc.get_sparse_core_info()
    ncores = info.num_cores
    nworkers = ncores * info.num_subcores
    rows_per_w = nq // nworkers
    nv = ncm // _L
    mesh = plsc.VectorSubcoreMesh(core_axis_name="c", subcore_axis_name="s")

    @functools.partial(
        pl.kernel,
        mesh=mesh,
        out_type=[
            jax.ShapeDtypeStruct((nq * _K,), jnp.float32),
            jax.ShapeDtypeStruct((nq * _K,), jnp.int32),
        ],
        scratch_types=[
            pltpu.VMEM((ncm,), jnp.float32),
            pltpu.VMEM((8, _SUB), jnp.float32),
            pltpu.VMEM((8, _SUB), jnp.float32),
            pltpu.SMEM((ncm + 16,), jnp.int32),
            pltpu.VMEM((_K,), jnp.float32),
            pltpu.VMEM((_K,), jnp.int32),
            pltpu.SemaphoreType.DMA,
            pltpu.SemaphoreType.DMA,
        ],
        compiler_params=pltpu.CompilerParams(needs_layout_passes=False),
    )
    def sc_kern(d_hbm, cm_hbm, ov_hbm, oi_hbm, cmb, dt0, dt1, clist,
                ovb, oib, sem0, sem1):
        wid = lax.axis_index("s") * ncores + lax.axis_index("c")
        lane = jax.lax.broadcasted_iota(jnp.int32, (_L,), 0)
        inf16 = jnp.full((_L,), jnp.inf, dtype=jnp.float32)

        def row_body(j, _):
            row = wid * rows_per_w + j
            rbase = (row // 8) * 8
            rr = row % 8
            pltpu.sync_copy(cm_hbm.at[pl.ds(row * ncm, ncm)], cmb)

            def u_vreg(i, uv):
                v = lax.sort(cmb[pl.ds(i * _L, _L)])
                merged = jnp.minimum(uv, lax.rev(v, (0,)))
                return lax.sort(merged)

            uvec = lax.fori_loop(0, nv, u_vreg, inf16)
            u = jnp.max(uvec)

            def cand_vreg(i, cnt):
                v = cmb[pl.ds(i * _L, _L)]
                cand = v <= u

                def c_cond(c):
                    _, msk = c
                    return jnp.any(msk)

                def c_body(c):
                    cn, msk = c
                    l = jnp.max(plsc.all_reduce_ffs(msk))
                    clist[cn] = i * _L + l
                    return (cn + 1, msk & ~(lane == l))

                cnt, _ = lax.while_loop(c_cond, c_body, (cnt, cand))
                return cnt

            ncand = lax.fori_loop(0, nv, cand_vreg, jnp.int32(0))

            def tile_src(ci):
                cid = clist[ci]
                return d_hbm.at[pl.ds(rbase, 8), pl.ds(cid * _SUB, _SUB)]

            pltpu.make_async_copy(tile_src(0), dt0, sem0).start()

            def scan_cand(ci, carry):
                rv, ri, thr = carry
                even = ci % 2 == 0

                @pl.when((ci + 1 < ncand) & even)
                def _():
                    pltpu.make_async_copy(tile_src(ci + 1), dt1,
                                          sem1).start()

                @pl.when((ci + 1 < ncand) & ~even)
                def _():
                    pltpu.make_async_copy(tile_src(ci + 1), dt0,
                                          sem0).start()

                @pl.when(even)
                def _():
                    pltpu.make_async_copy(tile_src(ci), dt0, sem0).wait()

                @pl.when(~even)
                def _():
                    pltpu.make_async_copy(tile_src(ci), dt1, sem1).wait()

                cid = clist[ci]

                def scan_vreg(t, carry2):
                    rv2, ri2, thr2 = carry2
                    sl = pl.ds(t * _L, _L)
                    dv = jnp.where(even, dt0[rr, sl], dt1[rr, sl])
                    gi = cid * _SUB + t * _L + lane
                    hit = dv <= thr2

                    def h_cond(c):
                        return jnp.any(c[3])

                    def h_body(c):
                        rv3, ri3, thr3, msk = c
                        l = jnp.max(plsc.all_reduce_ffs(msk))
                        sel = lane == l
                        ve = jnp.max(jnp.where(sel, dv, -jnp.inf))
                        ie = jnp.max(jnp.where(sel, gi, _IMIN))
                        ei = jnp.max(jnp.where(rv3 == thr3, ri3, _IMIN))
                        better = (ve < thr3) | ((ve == thr3) & (ie < ei))
                        selm = (rv3 == thr3) & (ri3 == ei) & better
                        rv3 = jnp.where(selm, ve, rv3)
                        ri3 = jnp.where(selm, ie, ri3)
                        return (rv3, ri3, jnp.max(rv3), msk & ~sel)

                    rv2, ri2, thr2, _ = lax.while_loop(
                        h_cond, h_body, (rv2, ri2, thr2, hit))
                    return (rv2, ri2, thr2)

                return lax.fori_loop(0, _SUB // _L, scan_vreg,
                                     (rv, ri, thr))

            rv, ri, _ = lax.fori_loop(0, ncand, scan_cand,
                                      (inf16, -(lane + 1), jnp.max(inf16)))

            ov = jnp.zeros((_L,), dtype=jnp.float32)
            oi = jnp.zeros((_L,), dtype=jnp.int32)
            for kk in range(_K):
                mval = jnp.min(rv)
                mi = jnp.min(jnp.where(rv == mval, ri, _IMAX))
                ksel = lane == kk
                ov = jnp.where(ksel, mval, ov)
                oi = jnp.where(ksel, mi, oi)
                rv = jnp.where((rv == mval) & (ri == mi), jnp.inf, rv)
            ovb[...] = ov
            oib[...] = oi
            pltpu.sync_copy(ovb, ov_hbm.at[pl.ds(row * _K, _K)])
            pltpu.sync_copy(oib, oi_hbm.at[pl.ds(row * _K, _K)])
            return 0

        lax.fori_loop(0, rows_per_w, row_body, 0)

    return sc_kern(d_hbm_arr, cm_flat)


def kernel(queries, keys):
    nq, dim = queries.shape
    nk = keys.shape[0]
    nc = pl.cdiv(nk, _CHUNK)
    npad = nc * _CHUNK - nk
    keys_p = jnp.pad(keys, ((0, npad), (0, 0))) if npad else keys
    d_hbm, cm3 = _tc_distances(queries, keys_p, nk)
    ncm = cm3.shape[0] * cm3.shape[2]
    cm = jnp.transpose(cm3, (1, 0, 2)).reshape(nq * ncm)
    ovf, oif = _sc_topk(d_hbm, cm, nq, ncm)
    return (ovf.reshape(nq, _K), oif.reshape(nq, _K))


# speedup vs baseline: 3.3356x; 1.0136x over previous
import functools

import jax
import jax.numpy as jnp
from jax import lax
from jax.experimental import pallas as pl
from jax.experimental.pallas import tpu as pltpu
from jax.experimental.pallas import tpu_sc as plsc

_K = 16
_CHUNK = 512
_SUB = 128
_L = 16
_IMAX = 2147483647
_IMIN = -2147483648


def _tc_body(nkeys, q_ref, k_ref, d_ref, cm_ref):
    c = pl.program_id(0)
    nq = q_ref.shape[0]
    cw = k_ref.shape[0]
    q = q_ref[...]
    k = k_ref[...]
    qn = jnp.sum(q * q, axis=1, keepdims=True)
    kn = jnp.sum(k * k, axis=1)[None, :]
    gidx = c * cw + jax.lax.broadcasted_iota(jnp.int32, (1, cw), 1)
    kn = jnp.where(gidx < nkeys, kn, jnp.inf)
    d = -2.0 * jnp.dot(q, k.T, preferred_element_type=jnp.float32)
    d = d + qn
    d = d + kn
    d_ref[...] = d
    cm_ref[...] = jnp.min(d.reshape(nq, cw // _SUB, _SUB),
                          axis=2)[None, :, :]


def _tc_distances(queries, keys_p, nkeys):
    nq, dim = queries.shape
    nkp = keys_p.shape[0]
    nc = nkp // _CHUNK
    return pl.pallas_call(
        functools.partial(_tc_body, nkeys),
        grid=(nc,),
        in_specs=[
            pl.BlockSpec((nq, dim), lambda c: (0, 0)),
            pl.BlockSpec((_CHUNK, dim), lambda c: (c, 0)),
        ],
        out_specs=[
            pl.BlockSpec((nq, _CHUNK), lambda c: (0, c)),
            pl.BlockSpec((1, nq, _CHUNK // _SUB), lambda c: (c, 0, 0)),
        ],
        out_shape=[
            jax.ShapeDtypeStruct((nq, nkp), jnp.float32),
            jax.ShapeDtypeStruct((nc, nq, _CHUNK // _SUB), jnp.float32),
        ],
        compiler_params=pltpu.CompilerParams(
            dimension_semantics=("arbitrary",)),
    )(queries, keys_p)


def _sc_topk(d_hbm_arr, cm_flat, nq, ncm):
    info = plsc.get_sparse_core_info()
    ncores = info.num_cores
    nworkers = ncores * info.num_subcores
    rows_per_w = nq // nworkers
    nv = ncm // _L
    mesh = plsc.VectorSubcoreMesh(core_axis_name="c", subcore_axis_name="s")

    @functools.partial(
        pl.kernel,
        mesh=mesh,
        out_type=[
            jax.ShapeDtypeStruct((nq * _K,), jnp.float32),
            jax.ShapeDtypeStruct((nq * _K,), jnp.int32),
        ],
        scratch_types=[
            pltpu.VMEM((ncm,), jnp.float32),
            pltpu.VMEM((ncm,), jnp.float32),
            pltpu.VMEM((8, _SUB), jnp.float32),
            pltpu.VMEM((8, _SUB), jnp.float32),
            pltpu.SMEM((ncm + 16,), jnp.int32),
            pltpu.VMEM((_K,), jnp.float32),
            pltpu.VMEM((_K,), jnp.int32),
            pltpu.SemaphoreType.DMA,
            pltpu.SemaphoreType.DMA,
            pltpu.SemaphoreType.DMA,
            pltpu.SemaphoreType.DMA,
        ],
        compiler_params=pltpu.CompilerParams(needs_layout_passes=False),
    )
    def sc_kern(d_hbm, cm_hbm, ov_hbm, oi_hbm, cmb0, cmb1, dt0, dt1, clist,
                ovb, oib, sem0, sem1, csem0, csem1):
        wid = lax.axis_index("s") * ncores + lax.axis_index("c")
        lane = jax.lax.broadcasted_iota(jnp.int32, (_L,), 0)
        inf16 = jnp.full((_L,), jnp.inf, dtype=jnp.float32)

        def cm_src(j):
            return cm_hbm.at[pl.ds((wid * rows_per_w + j) * ncm, ncm)]

        pltpu.make_async_copy(cm_src(0), cmb0, csem0).start()

        def row_body(j, _):
            row = wid * rows_per_w + j
            rbase = (row // 8) * 8
            rr = row % 8
            jeven = j % 2 == 0

            @pl.when((j + 1 < rows_per_w) & jeven)
            def _():
                pltpu.make_async_copy(cm_src(j + 1), cmb1, csem1).start()

            @pl.when((j + 1 < rows_per_w) & ~jeven)
            def _():
                pltpu.make_async_copy(cm_src(j + 1), cmb0, csem0).start()

            @pl.when(jeven)
            def _():
                pltpu.make_async_copy(cm_src(j), cmb0, csem0).wait()

            @pl.when(~jeven)
            def _():
                pltpu.make_async_copy(cm_src(j), cmb1, csem1).wait()

            def u_vreg(i, uv):
                sl = pl.ds(i * _L, _L)
                v = lax.sort(jnp.where(jeven, cmb0[sl], cmb1[sl]))
                merged = jnp.minimum(uv, lax.rev(v, (0,)))
                return lax.sort(merged)

            uvec = lax.fori_loop(0, nv, u_vreg, inf16)
            u = jnp.max(uvec)

            def cand_vreg(i, cnt):
                sl = pl.ds(i * _L, _L)
                v = jnp.where(jeven, cmb0[sl], cmb1[sl])
                cand = v <= u

                def c_cond(c):
                    _, msk = c
                    return jnp.any(msk)

                def c_body(c):
                    cn, msk = c
                    l = jnp.max(plsc.all_reduce_ffs(msk))
                    clist[cn] = i * _L + l
                    return (cn + 1, msk & ~(lane == l))

                cnt, _ = lax.while_loop(c_cond, c_body, (cnt, cand))
                return cnt

            ncand = lax.fori_loop(0, nv, cand_vreg, jnp.int32(0))

            def tile_src(ci):
                cid = clist[ci]
                return d_hbm.at[pl.ds(rbase, 8), pl.ds(cid * _SUB, _SUB)]

            pltpu.make_async_copy(tile_src(0), dt0, sem0).start()

            def scan_cand(ci, carry):
                rv, ri, thr = carry
                even = ci % 2 == 0

                @pl.when((ci + 1 < ncand) & even)
                def _():
                    pltpu.make_async_copy(tile_src(ci + 1), dt1,
                                          sem1).start()

                @pl.when((ci + 1 < ncand) & ~even)
                def _():
                    pltpu.make_async_copy(tile_src(ci + 1), dt0,
                                          sem0).start()

                @pl.when(even)
                def _():
                    pltpu.make_async_copy(tile_src(ci), dt0, sem0).wait()

                @pl.when(~even)
                def _():
                    pltpu.make_async_copy(tile_src(ci), dt1, sem1).wait()

                cid = clist[ci]

                def scan_vreg(t, carry2):
                    rv2, ri2, thr2 = carry2
                    sl = pl.ds(t * _L, _L)
                    dv = jnp.where(even, dt0[rr, sl], dt1[rr, sl])
                    gi = cid * _SUB + t * _L + lane
                    hit = dv <= thr2

                    def h_cond(c):
                        return jnp.any(c[3])

                    def h_body(c):
                        rv3, ri3, thr3, msk = c
                        l = jnp.max(plsc.all_reduce_ffs(msk))
                        sel = lane == l
                        ve = jnp.max(jnp.where(sel, dv, -jnp.inf))
                        ie = jnp.max(jnp.where(sel, gi, _IMIN))
                        ei = jnp.max(jnp.where(rv3 == thr3, ri3, _IMIN))
                        better = (ve < thr3) | ((ve == thr3) & (ie < ei))
                        selm = (rv3 == thr3) & (ri3 == ei) & better
                        rv3 = jnp.where(selm, ve, rv3)
                        ri3 = jnp.where(selm, ie, ri3)
                        return (rv3, ri3, jnp.max(rv3), msk & ~sel)

                    rv2, ri2, thr2, _ = lax.while_loop(
                        h_cond, h_body, (rv2, ri2, thr2, hit))
                    return (rv2, ri2, thr2)

                return lax.fori_loop(0, _SUB // _L, scan_vreg,
                                     (rv, ri, thr))

            rv, ri, _ = lax.fori_loop(0, ncand, scan_cand,
                                      (inf16, -(lane + 1), jnp.max(inf16)))

            ov = jnp.zeros((_L,), dtype=jnp.float32)
            oi = jnp.zeros((_L,), dtype=jnp.int32)
            for kk in range(_K):
                mval = jnp.min(rv)
                mi = jnp.min(jnp.where(rv == mval, ri, _IMAX))
                ksel = lane == kk
                ov = jnp.where(ksel, mval, ov)
                oi = jnp.where(ksel, mi, oi)
                rv = jnp.where((rv == mval) & (ri == mi), jnp.inf, rv)
            ovb[...] = ov
            oib[...] = oi
            pltpu.sync_copy(ovb, ov_hbm.at[pl.ds(row * _K, _K)])
            pltpu.sync_copy(oib, oi_hbm.at[pl.ds(row * _K, _K)])
            return 0

        lax.fori_loop(0, rows_per_w, row_body, 0)

    return sc_kern(d_hbm_arr, cm_flat)


def kernel(queries, keys):
    nq, dim = queries.shape
    nk = keys.shape[0]
    nc = pl.cdiv(nk, _CHUNK)
    npad = nc * _CHUNK - nk
    keys_p = jnp.pad(keys, ((0, npad), (0, 0))) if npad else keys
    d_hbm, cm3 = _tc_distances(queries, keys_p, nk)
    ncm = cm3.shape[0] * cm3.shape[2]
    cm = jnp.transpose(cm3, (1, 0, 2)).reshape(nq * ncm)
    ovf, oif = _sc_topk(d_hbm, cm, nq, ncm)
    return (ovf.reshape(nq, _K), oif.reshape(nq, _K))
